# baseline (speedup 1.0000x reference)
import functools

import jax
import jax.numpy as jnp
from jax import lax
from jax.experimental import pallas as pl
from jax.experimental.pallas import tpu as pltpu
from jax.experimental.pallas import tpu_sc as plsc

N_ROWS = 1000000
EMBED_DIM = 32
BATCH = 16384

NUM_WORKERS = 32
ROWS_PER_W = BATCH // NUM_WORKERS
CHUNK = 256
NCHUNK = ROWS_PER_W // CHUNK
QROWS = N_ROWS // 4

K0_COLS = 8192
K0_GRID = -(-N_ROWS // K0_COLS)
K0_OUT_ROWS = K0_COLS * EMBED_DIM // 128


def _repack_body(t_ref, o_ref):
    x = t_ref[...]
    pieces = [
        jnp.transpose(x[:, m * K0_OUT_ROWS:(m + 1) * K0_OUT_ROWS])
        for m in range(4)
    ]
    o_ref[...] = jnp.concatenate(pieces, axis=1)


def _repack(t):
    return pl.pallas_call(
        _repack_body,
        grid=(K0_GRID,),
        in_specs=[pl.BlockSpec((EMBED_DIM, K0_COLS), lambda i: (0, i))],
        out_specs=pl.BlockSpec((K0_OUT_ROWS, 128), lambda i: (i, 0)),
        out_shape=jax.ShapeDtypeStruct((K0_GRID * K0_OUT_ROWS, 128),
                                       jnp.float32),
    )(t)


def _sc_diff_body(qu_hbm, qi_hbm, users_hbm, pos_hbm, neg_hbm, out_hbm,
                  idx_u, idx_p, idx_n, q_u, q_p, q_n, off_u, off_p, off_n,
                  buf_u, buf_p, buf_n, diff, sem):
    wid = lax.axis_index("s") * 2 + lax.axis_index("c")
    base = wid * ROWS_PER_W

    tabs = ((idx_u, q_u, off_u, qu_hbm, buf_u),
            (idx_p, q_p, off_p, qi_hbm, buf_p),
            (idx_n, q_n, off_n, qi_hbm, buf_n))

    for src, (idx, _, _, _, _) in zip((users_hbm, pos_hbm, neg_hbm), tabs):
        pltpu.sync_copy(src.at[pl.ds(base, ROWS_PER_W)], idx)

    def prep_body(k, carry):
        for idx, q, off, _src, _buf in tabs:
            iv = idx[pl.ds(k * 16, 16)]
            q[pl.ds(k * 16, 16)] = iv >> 2
            off[pl.ds(k * 16, 16)] = (iv & 3) * 32
        return carry
    lax.fori_loop(0, ROWS_PER_W // 16, prep_body, 0)

    for h in range(NCHUNK):
        for _, q, _, src, buf in tabs:
            for c in range(CHUNK // 128):
                pltpu.async_copy(
                    src.at[q.at[pl.ds(h * CHUNK + c * 128, 128)]],
                    buf.at[pl.ds(c * 128, 128)], sem)
        for _, _, _, _, buf in tabs:
            pltpu.make_async_copy(qu_hbm.at[pl.ds(0, CHUNK)], buf, sem).wait()

        def dot_body(g, _):
            rows = g * 16 + lax.iota(jnp.int32, 16)
            pos0 = h * CHUNK + g * 16
            ou = off_u[pl.ds(pos0, 16)]
            op = off_p[pl.ds(pos0, 16)]
            on = off_n[pl.ds(pos0, 16)]
            acc = jnp.zeros((16,), jnp.float32)
            for d in range(EMBED_DIM):
                uu = plsc.load_gather(buf_u, [rows, ou + d])
                pp = plsc.load_gather(buf_p, [rows, op + d])
                nn = plsc.load_gather(buf_n, [rows, on + d])
                acc = acc + uu * (nn - pp)
            diff[pl.ds(pos0, 16)] = acc
            return _
        lax.fori_loop(0, CHUNK // 16, dot_body, 0)

    pltpu.sync_copy(diff, out_hbm.at[pl.ds(base, ROWS_PER_W)])


_sc_diff = functools.partial(
    pl.kernel,
    mesh=plsc.VectorSubcoreMesh(core_axis_name="c", subcore_axis_name="s"),
    out_type=jax.ShapeDtypeStruct((BATCH,), jnp.float32),
    compiler_params=pltpu.CompilerParams(needs_layout_passes=False),
    scratch_types=[
        pltpu.VMEM((ROWS_PER_W,), jnp.int32),
        pltpu.VMEM((ROWS_PER_W,), jnp.int32),
        pltpu.VMEM((ROWS_PER_W,), jnp.int32),
        pltpu.VMEM((ROWS_PER_W,), jnp.int32),
        pltpu.VMEM((ROWS_PER_W,), jnp.int32),
        pltpu.VMEM((ROWS_PER_W,), jnp.int32),
        pltpu.VMEM((ROWS_PER_W,), jnp.int32),
        pltpu.VMEM((ROWS_PER_W,), jnp.int32),
        pltpu.VMEM((ROWS_PER_W,), jnp.int32),
        pltpu.VMEM((CHUNK, 128), jnp.float32),
        pltpu.VMEM((CHUNK, 128), jnp.float32),
        pltpu.VMEM((CHUNK, 128), jnp.float32),
        pltpu.VMEM((ROWS_PER_W,), jnp.float32),
        pltpu.SemaphoreType.DMA,
    ],
)(_sc_diff_body)


def _loss_body(d_ref, o_ref):
    d = d_ref[...]
    sp = jnp.maximum(d, 0.0) + jnp.log1p(jnp.exp(-jnp.abs(d)))
    o_ref[...] = (jnp.sum(sp) * (1.0 / BATCH)).reshape(1, 1)


def kernel(embedding_user, embedding_item, users, pos, neg):
    qu = jnp.reshape(embedding_user, (QROWS, 128))
    qi = jnp.reshape(embedding_item, (QROWS, 128))
    users = users.astype(jnp.int32)
    pos = pos.astype(jnp.int32)
    neg = neg.astype(jnp.int32)
    diffs = _sc_diff(qu, qi, users, pos, neg)
    loss = pl.pallas_call(
        _loss_body,
        out_shape=jax.ShapeDtypeStruct((1, 1), jnp.float32),
    )(diffs.reshape(128, 128))
    return loss[0, 0]


# speedup vs baseline: 1.6825x; 1.6825x over previous
import functools

import jax
import jax.numpy as jnp
from jax import lax
from jax.experimental import pallas as pl
from jax.experimental.pallas import tpu as pltpu
from jax.experimental.pallas import tpu_sc as plsc

N_ROWS = 1000000
EMBED_DIM = 32
BATCH = 16384

NUM_WORKERS = 32
ROWS_PER_W = BATCH // NUM_WORKERS
CHUNK = 256
NCHUNK = ROWS_PER_W // CHUNK
QROWS = N_ROWS // 4

K0_COLS = 8192
K0_GRID = -(-N_ROWS // K0_COLS)
K0_OUT_ROWS = K0_COLS // 4
SHIFT_Q = K0_OUT_ROWS.bit_length() - 1
SHIFT_B = K0_COLS.bit_length() - 1


def _repack_body(t_ref, o_ref):
    x = t_ref[...]
    pieces = [
        jnp.transpose(x[:, m * K0_OUT_ROWS:(m + 1) * K0_OUT_ROWS])
        for m in range(4)
    ]
    o_ref[...] = jnp.concatenate(pieces, axis=1)


def _repack(t):
    return pl.pallas_call(
        _repack_body,
        grid=(K0_GRID,),
        in_specs=[pl.BlockSpec((EMBED_DIM, K0_COLS), lambda i: (0, i))],
        out_specs=pl.BlockSpec((K0_OUT_ROWS, 128), lambda i: (i, 0)),
        out_shape=jax.ShapeDtypeStruct((K0_GRID * K0_OUT_ROWS, 128),
                                       jnp.float32),
    )(t)


def _sc_diff_body(qu_hbm, qi_hbm, users_hbm, pos_hbm, neg_hbm, out_hbm,
                  idx_u, idx_p, idx_n, q_u, q_p, q_n, off_u, off_p, off_n,
                  buf_u, buf_p, buf_n, diff, sem):
    wid = lax.axis_index("s") * 2 + lax.axis_index("c")
    base = wid * ROWS_PER_W

    tabs = ((idx_u, q_u, off_u, qu_hbm, buf_u),
            (idx_p, q_p, off_p, qi_hbm, buf_p),
            (idx_n, q_n, off_n, qi_hbm, buf_n))

    for src, (idx, _, _, _, _) in zip((users_hbm, pos_hbm, neg_hbm), tabs):
        pltpu.sync_copy(src.at[pl.ds(base, ROWS_PER_W)], idx)

    def prep_body(k, carry):
        for idx, q, off, _src, _buf in tabs:
            iv = idx[pl.ds(k * 16, 16)]
            q[pl.ds(k * 16, 16)] = ((iv >> SHIFT_B) << SHIFT_Q) + (
                iv & (K0_OUT_ROWS - 1))
            off[pl.ds(k * 16, 16)] = ((iv >> SHIFT_Q) & 3) * 32
        return carry
    lax.fori_loop(0, ROWS_PER_W // 16, prep_body, 0)

    for h in range(NCHUNK):
        for _, q, _, src, buf in tabs:
            for c in range(CHUNK // 128):
                pltpu.async_copy(
                    src.at[q.at[pl.ds(h * CHUNK + c * 128, 128)]],
                    buf.at[pl.ds(c * 128, 128)], sem)
        for _, _, _, _, buf in tabs:
            pltpu.make_async_copy(qu_hbm.at[pl.ds(0, CHUNK)], buf, sem).wait()

        def dot_body(g, _):
            rows = g * 16 + lax.iota(jnp.int32, 16)
            pos0 = h * CHUNK + g * 16
            ou = off_u[pl.ds(pos0, 16)]
            op = off_p[pl.ds(pos0, 16)]
            on = off_n[pl.ds(pos0, 16)]
            acc = jnp.zeros((16,), jnp.float32)
            for d in range(EMBED_DIM):
                uu = plsc.load_gather(buf_u, [rows, ou + d])
                pp = plsc.load_gather(buf_p, [rows, op + d])
                nn = plsc.load_gather(buf_n, [rows, on + d])
                acc = acc + uu * (nn - pp)
            diff[pl.ds(pos0, 16)] = acc
            return _
        lax.fori_loop(0, CHUNK // 16, dot_body, 0)

    pltpu.sync_copy(diff, out_hbm.at[pl.ds(base, ROWS_PER_W)])


_sc_diff = functools.partial(
    pl.kernel,
    mesh=plsc.VectorSubcoreMesh(core_axis_name="c", subcore_axis_name="s"),
    out_type=jax.ShapeDtypeStruct((BATCH,), jnp.float32),
    compiler_params=pltpu.CompilerParams(needs_layout_passes=False),
    scratch_types=[
        pltpu.VMEM((ROWS_PER_W,), jnp.int32),
        pltpu.VMEM((ROWS_PER_W,), jnp.int32),
        pltpu.VMEM((ROWS_PER_W,), jnp.int32),
        pltpu.VMEM((ROWS_PER_W,), jnp.int32),
        pltpu.VMEM((ROWS_PER_W,), jnp.int32),
        pltpu.VMEM((ROWS_PER_W,), jnp.int32),
        pltpu.VMEM((ROWS_PER_W,), jnp.int32),
        pltpu.VMEM((ROWS_PER_W,), jnp.int32),
        pltpu.VMEM((ROWS_PER_W,), jnp.int32),
        pltpu.VMEM((CHUNK, 128), jnp.float32),
        pltpu.VMEM((CHUNK, 128), jnp.float32),
        pltpu.VMEM((CHUNK, 128), jnp.float32),
        pltpu.VMEM((ROWS_PER_W,), jnp.float32),
        pltpu.SemaphoreType.DMA,
    ],
)(_sc_diff_body)


def _loss_body(d_ref, o_ref):
    d = d_ref[...]
    sp = jnp.maximum(d, 0.0) + jnp.log1p(jnp.exp(-jnp.abs(d)))
    o_ref[...] = (jnp.sum(sp) * (1.0 / BATCH)).reshape(1, 1)


def kernel(embedding_user, embedding_item, users, pos, neg):
    qu = _repack(embedding_user.T)
    qi = _repack(embedding_item.T)
    users = users.astype(jnp.int32)
    pos = pos.astype(jnp.int32)
    neg = neg.astype(jnp.int32)
    diffs = _sc_diff(qu, qi, users, pos, neg)
    loss = pl.pallas_call(
        _loss_body,
        out_shape=jax.ShapeDtypeStruct((1, 1), jnp.float32),
    )(diffs.reshape(128, 128))
    return loss[0, 0]


# speedup vs baseline: 1.7064x; 1.0142x over previous
import functools

import jax
import jax.numpy as jnp
from jax import lax
from jax.experimental import pallas as pl
from jax.experimental.pallas import tpu as pltpu
from jax.experimental.pallas import tpu_sc as plsc

N_ROWS = 1000000
EMBED_DIM = 32
BATCH = 16384

NUM_WORKERS = 32
ROWS_PER_W = BATCH // NUM_WORKERS
CHUNK = 256
NCHUNK = ROWS_PER_W // CHUNK
QROWS = N_ROWS // 4

K0_COLS = 16384
K0_GRID = -(-N_ROWS // K0_COLS)
K0_OUT_ROWS = K0_COLS // 4
SHIFT_Q = K0_OUT_ROWS.bit_length() - 1
SHIFT_B = K0_COLS.bit_length() - 1


def _repack_body(t_ref, o_ref):
    x = t_ref[...]
    pieces = [
        jnp.transpose(x[:, m * K0_OUT_ROWS:(m + 1) * K0_OUT_ROWS])
        for m in range(4)
    ]
    o_ref[...] = jnp.concatenate(pieces, axis=1)


def _repack(t):
    return pl.pallas_call(
        _repack_body,
        grid=(K0_GRID,),
        in_specs=[pl.BlockSpec((EMBED_DIM, K0_COLS), lambda i: (0, i))],
        out_specs=pl.BlockSpec((K0_OUT_ROWS, 128), lambda i: (i, 0)),
        out_shape=jax.ShapeDtypeStruct((K0_GRID * K0_OUT_ROWS, 128),
                                       jnp.float32),
    )(t)


def _sc_diff_body(qu_hbm, qi_hbm, users_hbm, pos_hbm, neg_hbm, out_hbm,
                  idx_u, idx_p, idx_n, q_u, q_p, q_n, off_u, off_p, off_n,
                  buf_u, buf_p, buf_n, diff, sem):
    wid = lax.axis_index("s") * 2 + lax.axis_index("c")
    base = wid * ROWS_PER_W

    tabs = ((idx_u, q_u, off_u, qu_hbm, buf_u),
            (idx_p, q_p, off_p, qi_hbm, buf_p),
            (idx_n, q_n, off_n, qi_hbm, buf_n))

    for src, (idx, _, _, _, _) in zip((users_hbm, pos_hbm, neg_hbm), tabs):
        pltpu.sync_copy(src.at[pl.ds(base, ROWS_PER_W)], idx)

    def prep_body(k, carry):
        for idx, q, off, _src, _buf in tabs:
            iv = idx[pl.ds(k * 16, 16)]
            q[pl.ds(k * 16, 16)] = ((iv >> SHIFT_B) << SHIFT_Q) + (
                iv & (K0_OUT_ROWS - 1))
            off[pl.ds(k * 16, 16)] = ((iv >> SHIFT_Q) & 3) * 32
        return carry
    lax.fori_loop(0, ROWS_PER_W // 16, prep_body, 0)

    for h in range(NCHUNK):
        for _, q, _, src, buf in tabs:
            for c in range(CHUNK // 128):
                pltpu.async_copy(
                    src.at[q.at[pl.ds(h * CHUNK + c * 128, 128)]],
                    buf.at[pl.ds(c * 128, 128)], sem)
        for _, _, _, _, buf in tabs:
            pltpu.make_async_copy(qu_hbm.at[pl.ds(0, CHUNK)], buf, sem).wait()

        def dot_body(g, _):
            rows = g * 16 + lax.iota(jnp.int32, 16)
            pos0 = h * CHUNK + g * 16
            ou = off_u[pl.ds(pos0, 16)]
            op = off_p[pl.ds(pos0, 16)]
            on = off_n[pl.ds(pos0, 16)]
            acc = jnp.zeros((16,), jnp.float32)
            for d in range(EMBED_DIM):
                uu = plsc.load_gather(buf_u, [rows, ou + d])
                pp = plsc.load_gather(buf_p, [rows, op + d])
                nn = plsc.load_gather(buf_n, [rows, on + d])
                acc = acc + uu * (nn - pp)
            diff[pl.ds(pos0, 16)] = acc
            return _
        lax.fori_loop(0, CHUNK // 16, dot_body, 0)

    pltpu.sync_copy(diff, out_hbm.at[pl.ds(base, ROWS_PER_W)])


_sc_diff = functools.partial(
    pl.kernel,
    mesh=plsc.VectorSubcoreMesh(core_axis_name="c", subcore_axis_name="s"),
    out_type=jax.ShapeDtypeStruct((BATCH,), jnp.float32),
    compiler_params=pltpu.CompilerParams(needs_layout_passes=False),
    scratch_types=[
        pltpu.VMEM((ROWS_PER_W,), jnp.int32),
        pltpu.VMEM((ROWS_PER_W,), jnp.int32),
        pltpu.VMEM((ROWS_PER_W,), jnp.int32),
        pltpu.VMEM((ROWS_PER_W,), jnp.int32),
        pltpu.VMEM((ROWS_PER_W,), jnp.int32),
        pltpu.VMEM((ROWS_PER_W,), jnp.int32),
        pltpu.VMEM((ROWS_PER_W,), jnp.int32),
        pltpu.VMEM((ROWS_PER_W,), jnp.int32),
        pltpu.VMEM((ROWS_PER_W,), jnp.int32),
        pltpu.VMEM((CHUNK, 128), jnp.float32),
        pltpu.VMEM((CHUNK, 128), jnp.float32),
        pltpu.VMEM((CHUNK, 128), jnp.float32),
        pltpu.VMEM((ROWS_PER_W,), jnp.float32),
        pltpu.SemaphoreType.DMA,
    ],
)(_sc_diff_body)


def _loss_body(d_ref, o_ref):
    d = d_ref[...]
    sp = jnp.maximum(d, 0.0) + jnp.log1p(jnp.exp(-jnp.abs(d)))
    o_ref[...] = (jnp.sum(sp) * (1.0 / BATCH)).reshape(1, 1)


def kernel(embedding_user, embedding_item, users, pos, neg):
    qu = _repack(embedding_user.T)
    qi = _repack(embedding_item.T)
    users = users.astype(jnp.int32)
    pos = pos.astype(jnp.int32)
    neg = neg.astype(jnp.int32)
    diffs = _sc_diff(qu, qi, users, pos, neg)
    loss = pl.pallas_call(
        _loss_body,
        out_shape=jax.ShapeDtypeStruct((1, 1), jnp.float32),
    )(diffs.reshape(128, 128))
    return loss[0, 0]


# speedup vs baseline: 1.7498x; 1.0254x over previous
import functools

import jax
import jax.numpy as jnp
from jax import lax
from jax.experimental import pallas as pl
from jax.experimental.pallas import tpu as pltpu
from jax.experimental.pallas import tpu_sc as plsc

N_ROWS = 1000000
EMBED_DIM = 32
BATCH = 16384

NUM_WORKERS = 32
ROWS_PER_W = BATCH // NUM_WORKERS
CHUNK = 128
NCHUNK = ROWS_PER_W // CHUNK
QROWS = N_ROWS // 4

K0_COLS = 16384
K0_GRID = -(-N_ROWS // K0_COLS)
K0_OUT_ROWS = K0_COLS // 4
SHIFT_Q = K0_OUT_ROWS.bit_length() - 1
SHIFT_B = K0_COLS.bit_length() - 1


def _pack_block(x):
    pieces = [
        jnp.transpose(x[:, m * K0_OUT_ROWS:(m + 1) * K0_OUT_ROWS])
        for m in range(4)
    ]
    return jnp.concatenate(pieces, axis=1)


def _repack_body(u_ref, i_ref, ou_ref, oi_ref):
    ou_ref[...] = _pack_block(u_ref[...])
    oi_ref[...] = _pack_block(i_ref[...])


def _repack(ut, it):
    spec_in = pl.BlockSpec((EMBED_DIM, K0_COLS), lambda i: (0, i))
    spec_out = pl.BlockSpec((K0_OUT_ROWS, 128), lambda i: (i, 0))
    packed_t = jax.ShapeDtypeStruct((K0_GRID * K0_OUT_ROWS, 128),
                                    jnp.float32)
    return pl.pallas_call(
        _repack_body,
        grid=(K0_GRID,),
        in_specs=[spec_in, spec_in],
        out_specs=[spec_out, spec_out],
        out_shape=[packed_t, packed_t],
    )(ut, it)


def _sc_diff_body(qu_hbm, qi_hbm, users_hbm, pos_hbm, neg_hbm, out_hbm,
                  idx_u, idx_p, idx_n, q_u, q_p, q_n, off_u, off_p, off_n,
                  buf_u0, buf_u1, buf_p0, buf_p1, buf_n0, buf_n1, diff,
                  sem0, sem1):
    wid = lax.axis_index("s") * 2 + lax.axis_index("c")
    base = wid * ROWS_PER_W

    sems = (sem0, sem1)
    tabs = ((idx_u, q_u, off_u, qu_hbm, (buf_u0, buf_u1)),
            (idx_p, q_p, off_p, qi_hbm, (buf_p0, buf_p1)),
            (idx_n, q_n, off_n, qi_hbm, (buf_n0, buf_n1)))

    for src, (idx, _, _, _, _) in zip((users_hbm, pos_hbm, neg_hbm), tabs):
        pltpu.sync_copy(src.at[pl.ds(base, ROWS_PER_W)], idx)

    def prep_body(k, carry):
        for idx, q, off, _src, _buf in tabs:
            iv = idx[pl.ds(k * 16, 16)]
            q[pl.ds(k * 16, 16)] = ((iv >> SHIFT_B) << SHIFT_Q) + (
                iv & (K0_OUT_ROWS - 1))
            off[pl.ds(k * 16, 16)] = ((iv >> SHIFT_Q) & 3) * 32
        return carry
    lax.fori_loop(0, ROWS_PER_W // 16, prep_body, 0)

    def fire(h, s):
        for _, q, _, src, bufs in tabs:
            pltpu.async_copy(src.at[q.at[pl.ds(h * CHUNK, CHUNK)]],
                             bufs[s], sems[s])

    def drain(s):
        for _, _, _, _, bufs in tabs:
            pltpu.make_async_copy(qu_hbm.at[pl.ds(0, CHUNK)],
                                  bufs[s], sems[s]).wait()

    fire(0, 0)
    for h in range(NCHUNK):
        s = h % 2
        if h + 1 < NCHUNK:
            fire(h + 1, 1 - s)
        drain(s)

        def dot_body(g, _, h=h, s=s):
            rows = g * 16 + lax.iota(jnp.int32, 16)
            pos0 = h * CHUNK + g * 16
            ou = off_u[pl.ds(pos0, 16)]
            op = off_p[pl.ds(pos0, 16)]
            on = off_n[pl.ds(pos0, 16)]
            acc = jnp.zeros((16,), jnp.float32)
            for d in range(EMBED_DIM):
                uu = plsc.load_gather(tabs[0][4][s], [rows, ou + d])
                pp = plsc.load_gather(tabs[1][4][s], [rows, op + d])
                nn = plsc.load_gather(tabs[2][4][s], [rows, on + d])
                acc = acc + uu * (nn - pp)
            diff[pl.ds(pos0, 16)] = acc
            return _
        lax.fori_loop(0, CHUNK // 16, dot_body, 0)

    pltpu.sync_copy(diff, out_hbm.at[pl.ds(base, ROWS_PER_W)])


_sc_diff = functools.partial(
    pl.kernel,
    mesh=plsc.VectorSubcoreMesh(core_axis_name="c", subcore_axis_name="s"),
    out_type=jax.ShapeDtypeStruct((BATCH,), jnp.float32),
    compiler_params=pltpu.CompilerParams(needs_layout_passes=False),
    scratch_types=[
        pltpu.VMEM((ROWS_PER_W,), jnp.int32),
        pltpu.VMEM((ROWS_PER_W,), jnp.int32),
        pltpu.VMEM((ROWS_PER_W,), jnp.int32),
        pltpu.VMEM((ROWS_PER_W,), jnp.int32),
        pltpu.VMEM((ROWS_PER_W,), jnp.int32),
        pltpu.VMEM((ROWS_PER_W,), jnp.int32),
        pltpu.VMEM((ROWS_PER_W,), jnp.int32),
        pltpu.VMEM((ROWS_PER_W,), jnp.int32),
        pltpu.VMEM((ROWS_PER_W,), jnp.int32),
        pltpu.VMEM((CHUNK, 128), jnp.float32),
        pltpu.VMEM((CHUNK, 128), jnp.float32),
        pltpu.VMEM((CHUNK, 128), jnp.float32),
        pltpu.VMEM((CHUNK, 128), jnp.float32),
        pltpu.VMEM((CHUNK, 128), jnp.float32),
        pltpu.VMEM((CHUNK, 128), jnp.float32),
        pltpu.VMEM((ROWS_PER_W,), jnp.float32),
        pltpu.SemaphoreType.DMA,
        pltpu.SemaphoreType.DMA,
    ],
)(_sc_diff_body)


def _loss_body(d_ref, o_ref):
    d = d_ref[...]
    sp = jnp.maximum(d, 0.0) + jnp.log1p(jnp.exp(-jnp.abs(d)))
    o_ref[...] = (jnp.sum(sp) * (1.0 / BATCH)).reshape(1, 1)


def kernel(embedding_user, embedding_item, users, pos, neg):
    qu, qi = _repack(embedding_user.T, embedding_item.T)
    users = users.astype(jnp.int32)
    pos = pos.astype(jnp.int32)
    neg = neg.astype(jnp.int32)
    diffs = _sc_diff(qu, qi, users, pos, neg)
    loss = pl.pallas_call(
        _loss_body,
        out_shape=jax.ShapeDtypeStruct((1, 1), jnp.float32),
    )(diffs.reshape(128, 128))
    return loss[0, 0]


# speedup vs baseline: 1.7610x; 1.0064x over previous
import functools

import jax
import jax.numpy as jnp
from jax import lax
from jax.experimental import pallas as pl
from jax.experimental.pallas import tpu as pltpu
from jax.experimental.pallas import tpu_sc as plsc

N_ROWS = 1000000
EMBED_DIM = 32
BATCH = 16384

NUM_WORKERS = 32
ROWS_PER_W = BATCH // NUM_WORKERS
CHUNK = 128
NCHUNK = ROWS_PER_W // CHUNK
QROWS = N_ROWS // 4

K0_COLS = 16384
K0_GRID = -(-N_ROWS // K0_COLS)
K0_OUT_ROWS = K0_COLS // 4
SHIFT_Q = K0_OUT_ROWS.bit_length() - 1
SHIFT_B = K0_COLS.bit_length() - 1


def _repack_body(u_ref, i_ref, ou_ref, oi_ref):
    for t_ref, o_ref in ((u_ref, ou_ref), (i_ref, oi_ref)):
        for m in range(4):
            piece = jnp.transpose(
                t_ref[:, m * K0_OUT_ROWS:(m + 1) * K0_OUT_ROWS])
            o_ref[:, m * EMBED_DIM:(m + 1) * EMBED_DIM] = piece


def _repack(ut, it):
    spec_in = pl.BlockSpec((EMBED_DIM, K0_COLS), lambda i: (0, i))
    spec_out = pl.BlockSpec((K0_OUT_ROWS, 128), lambda i: (i, 0))
    packed_t = jax.ShapeDtypeStruct((K0_GRID * K0_OUT_ROWS, 128),
                                    jnp.float32)
    return pl.pallas_call(
        _repack_body,
        grid=(K0_GRID,),
        in_specs=[spec_in, spec_in],
        out_specs=[spec_out, spec_out],
        out_shape=[packed_t, packed_t],
    )(ut, it)


def _sc_diff_body(qu_hbm, qi_hbm, users_hbm, pos_hbm, neg_hbm, out_hbm,
                  idx_u, idx_p, idx_n, q_u, q_p, q_n, off_u, off_p, off_n,
                  buf_u0, buf_u1, buf_p0, buf_p1, buf_n0, buf_n1, diff,
                  sem0, sem1):
    wid = lax.axis_index("s") * 2 + lax.axis_index("c")
    base = wid * ROWS_PER_W

    sems = (sem0, sem1)
    tabs = ((idx_u, q_u, off_u, qu_hbm, (buf_u0, buf_u1)),
            (idx_p, q_p, off_p, qi_hbm, (buf_p0, buf_p1)),
            (idx_n, q_n, off_n, qi_hbm, (buf_n0, buf_n1)))

    for src, (idx, _, _, _, _) in zip((users_hbm, pos_hbm, neg_hbm), tabs):
        pltpu.sync_copy(src.at[pl.ds(base, ROWS_PER_W)], idx)

    def prep_body(k, carry):
        for idx, q, off, _src, _buf in tabs:
            iv = idx[pl.ds(k * 16, 16)]
            q[pl.ds(k * 16, 16)] = ((iv >> SHIFT_B) << SHIFT_Q) + (
                iv & (K0_OUT_ROWS - 1))
            off[pl.ds(k * 16, 16)] = ((iv >> SHIFT_Q) & 3) * 32
        return carry
    lax.fori_loop(0, ROWS_PER_W // 16, prep_body, 0)

    def fire(h, s):
        for _, q, _, src, bufs in tabs:
            pltpu.async_copy(src.at[q.at[pl.ds(h * CHUNK, CHUNK)]],
                             bufs[s], sems[s])

    def drain(s):
        for _, _, _, _, bufs in tabs:
            pltpu.make_async_copy(qu_hbm.at[pl.ds(0, CHUNK)],
                                  bufs[s], sems[s]).wait()

    fire(0, 0)
    for h in range(NCHUNK):
        s = h % 2
        if h + 1 < NCHUNK:
            fire(h + 1, 1 - s)
        drain(s)

        def dot_body(g, _, h=h, s=s):
            rows = g * 16 + lax.iota(jnp.int32, 16)
            pos0 = h * CHUNK + g * 16
            ou = off_u[pl.ds(pos0, 16)]
            op = off_p[pl.ds(pos0, 16)]
            on = off_n[pl.ds(pos0, 16)]
            accs = [jnp.zeros((16,), jnp.float32) for _ in range(4)]
            for d in range(EMBED_DIM):
                uu = plsc.load_gather(tabs[0][4][s], [rows, ou + d])
                pp = plsc.load_gather(tabs[1][4][s], [rows, op + d])
                nn = plsc.load_gather(tabs[2][4][s], [rows, on + d])
                accs[d % 4] = accs[d % 4] + uu * (nn - pp)
            diff[pl.ds(pos0, 16)] = (accs[0] + accs[1]) + (accs[2] + accs[3])
            return _
        lax.fori_loop(0, CHUNK // 16, dot_body, 0)

    pltpu.sync_copy(diff, out_hbm.at[pl.ds(base, ROWS_PER_W)])


_sc_diff = functools.partial(
    pl.kernel,
    mesh=plsc.VectorSubcoreMesh(core_axis_name="c", subcore_axis_name="s"),
    out_type=jax.ShapeDtypeStruct((BATCH,), jnp.float32),
    compiler_params=pltpu.CompilerParams(needs_layout_passes=False),
    scratch_types=[
        pltpu.VMEM((ROWS_PER_W,), jnp.int32),
        pltpu.VMEM((ROWS_PER_W,), jnp.int32),
        pltpu.VMEM((ROWS_PER_W,), jnp.int32),
        pltpu.VMEM((ROWS_PER_W,), jnp.int32),
        pltpu.VMEM((ROWS_PER_W,), jnp.int32),
        pltpu.VMEM((ROWS_PER_W,), jnp.int32),
        pltpu.VMEM((ROWS_PER_W,), jnp.int32),
        pltpu.VMEM((ROWS_PER_W,), jnp.int32),
        pltpu.VMEM((ROWS_PER_W,), jnp.int32),
        pltpu.VMEM((CHUNK, 128), jnp.float32),
        pltpu.VMEM((CHUNK, 128), jnp.float32),
        pltpu.VMEM((CHUNK, 128), jnp.float32),
        pltpu.VMEM((CHUNK, 128), jnp.float32),
        pltpu.VMEM((CHUNK, 128), jnp.float32),
        pltpu.VMEM((CHUNK, 128), jnp.float32),
        pltpu.VMEM((ROWS_PER_W,), jnp.float32),
        pltpu.SemaphoreType.DMA,
        pltpu.SemaphoreType.DMA,
    ],
)(_sc_diff_body)


def _loss_body(d_ref, o_ref):
    d = d_ref[...]
    sp = jnp.maximum(d, 0.0) + jnp.log1p(jnp.exp(-jnp.abs(d)))
    o_ref[...] = (jnp.sum(sp) * (1.0 / BATCH)).reshape(1, 1)


def kernel(embedding_user, embedding_item, users, pos, neg):
    qu, qi = _repack(embedding_user.T, embedding_item.T)
    users = users.astype(jnp.int32)
    pos = pos.astype(jnp.int32)
    neg = neg.astype(jnp.int32)
    diffs = _sc_diff(qu, qi, users, pos, neg)
    loss = pl.pallas_call(
        _loss_body,
        out_shape=jax.ShapeDtypeStruct((1, 1), jnp.float32),
    )(diffs.reshape(128, 128))
    return loss[0, 0]


# speedup vs baseline: 1.7635x; 1.0014x over previous
import functools

import jax
import jax.numpy as jnp
from jax import lax
from jax.experimental import pallas as pl
from jax.experimental.pallas import tpu as pltpu
from jax.experimental.pallas import tpu_sc as plsc

N_ROWS = 1000000
EMBED_DIM = 32
BATCH = 16384

NUM_WORKERS = 32
ROWS_PER_W = BATCH // NUM_WORKERS
CHUNK = 128
NCHUNK = ROWS_PER_W // CHUNK
QROWS = N_ROWS // 4

K0_COLS = 32768
K0_GRID = -(-N_ROWS // K0_COLS)
K0_OUT_ROWS = K0_COLS // 4
SHIFT_Q = K0_OUT_ROWS.bit_length() - 1
SHIFT_B = K0_COLS.bit_length() - 1


def _repack_body(u_ref, i_ref, ou_ref, oi_ref):
    for t_ref, o_ref in ((u_ref, ou_ref), (i_ref, oi_ref)):
        for m in range(4):
            piece = jnp.transpose(
                t_ref[:, m * K0_OUT_ROWS:(m + 1) * K0_OUT_ROWS])
            o_ref[:, m * EMBED_DIM:(m + 1) * EMBED_DIM] = piece


def _repack(ut, it):
    spec_in = pl.BlockSpec((EMBED_DIM, K0_COLS), lambda i: (0, i))
    spec_out = pl.BlockSpec((K0_OUT_ROWS, 128), lambda i: (i, 0))
    packed_t = jax.ShapeDtypeStruct((K0_GRID * K0_OUT_ROWS, 128),
                                    jnp.float32)
    return pl.pallas_call(
        _repack_body,
        grid=(K0_GRID,),
        in_specs=[spec_in, spec_in],
        out_specs=[spec_out, spec_out],
        out_shape=[packed_t, packed_t],
    )(ut, it)


def _sc_diff_body(qu_hbm, qi_hbm, users_hbm, pos_hbm, neg_hbm, out_hbm,
                  idx_u, idx_p, idx_n, q_u, q_p, q_n, off_u, off_p, off_n,
                  buf_u0, buf_u1, buf_p0, buf_p1, buf_n0, buf_n1, diff,
                  sem0, sem1):
    wid = lax.axis_index("s") * 2 + lax.axis_index("c")
    base = wid * ROWS_PER_W

    sems = (sem0, sem1)
    tabs = ((idx_u, q_u, off_u, qu_hbm, (buf_u0, buf_u1)),
            (idx_p, q_p, off_p, qi_hbm, (buf_p0, buf_p1)),
            (idx_n, q_n, off_n, qi_hbm, (buf_n0, buf_n1)))

    for src, (idx, _, _, _, _) in zip((users_hbm, pos_hbm, neg_hbm), tabs):
        pltpu.sync_copy(src.at[pl.ds(base, ROWS_PER_W)], idx)

    def prep_body(k, carry):
        for idx, q, off, _src, _buf in tabs:
            iv = idx[pl.ds(k * 16, 16)]
            q[pl.ds(k * 16, 16)] = ((iv >> SHIFT_B) << SHIFT_Q) + (
                iv & (K0_OUT_ROWS - 1))
            off[pl.ds(k * 16, 16)] = ((iv >> SHIFT_Q) & 3) * 32
        return carry
    lax.fori_loop(0, ROWS_PER_W // 16, prep_body, 0)

    def fire(h, s):
        for _, q, _, src, bufs in tabs:
            pltpu.async_copy(src.at[q.at[pl.ds(h * CHUNK, CHUNK)]],
                             bufs[s], sems[s])

    def drain(s):
        for _, _, _, _, bufs in tabs:
            pltpu.make_async_copy(qu_hbm.at[pl.ds(0, CHUNK)],
                                  bufs[s], sems[s]).wait()

    fire(0, 0)
    for h in range(NCHUNK):
        s = h % 2
        if h + 1 < NCHUNK:
            fire(h + 1, 1 - s)
        drain(s)

        def dot_body(g, _, h=h, s=s):
            rows = g * 16 + lax.iota(jnp.int32, 16)
            pos0 = h * CHUNK + g * 16
            ou = off_u[pl.ds(pos0, 16)]
            op = off_p[pl.ds(pos0, 16)]
            on = off_n[pl.ds(pos0, 16)]
            accs = [jnp.zeros((16,), jnp.float32) for _ in range(4)]
            for d in range(EMBED_DIM):
                uu = plsc.load_gather(tabs[0][4][s], [rows, ou + d])
                pp = plsc.load_gather(tabs[1][4][s], [rows, op + d])
                nn = plsc.load_gather(tabs[2][4][s], [rows, on + d])
                accs[d % 4] = accs[d % 4] + uu * (nn - pp)
            diff[pl.ds(pos0, 16)] = (accs[0] + accs[1]) + (accs[2] + accs[3])
            return _
        lax.fori_loop(0, CHUNK // 16, dot_body, 0)

    pltpu.sync_copy(diff, out_hbm.at[pl.ds(base, ROWS_PER_W)])


_sc_diff = functools.partial(
    pl.kernel,
    mesh=plsc.VectorSubcoreMesh(core_axis_name="c", subcore_axis_name="s"),
    out_type=jax.ShapeDtypeStruct((BATCH,), jnp.float32),
    compiler_params=pltpu.CompilerParams(needs_layout_passes=False),
    scratch_types=[
        pltpu.VMEM((ROWS_PER_W,), jnp.int32),
        pltpu.VMEM((ROWS_PER_W,), jnp.int32),
        pltpu.VMEM((ROWS_PER_W,), jnp.int32),
        pltpu.VMEM((ROWS_PER_W,), jnp.int32),
        pltpu.VMEM((ROWS_PER_W,), jnp.int32),
        pltpu.VMEM((ROWS_PER_W,), jnp.int32),
        pltpu.VMEM((ROWS_PER_W,), jnp.int32),
        pltpu.VMEM((ROWS_PER_W,), jnp.int32),
        pltpu.VMEM((ROWS_PER_W,), jnp.int32),
        pltpu.VMEM((CHUNK, 128), jnp.float32),
        pltpu.VMEM((CHUNK, 128), jnp.float32),
        pltpu.VMEM((CHUNK, 128), jnp.float32),
        pltpu.VMEM((CHUNK, 128), jnp.float32),
        pltpu.VMEM((CHUNK, 128), jnp.float32),
        pltpu.VMEM((CHUNK, 128), jnp.float32),
        pltpu.VMEM((ROWS_PER_W,), jnp.float32),
        pltpu.SemaphoreType.DMA,
        pltpu.SemaphoreType.DMA,
    ],
)(_sc_diff_body)


def _loss_body(d_ref, o_ref):
    d = d_ref[...]
    sp = jnp.maximum(d, 0.0) + jnp.log1p(jnp.exp(-jnp.abs(d)))
    o_ref[...] = (jnp.sum(sp) * (1.0 / BATCH)).reshape(1, 1)


def kernel(embedding_user, embedding_item, users, pos, neg):
    qu, qi = _repack(embedding_user.T, embedding_item.T)
    users = users.astype(jnp.int32)
    pos = pos.astype(jnp.int32)
    neg = neg.astype(jnp.int32)
    diffs = _sc_diff(qu, qi, users, pos, neg)
    loss = pl.pallas_call(
        _loss_body,
        out_shape=jax.ShapeDtypeStruct((1, 1), jnp.float32),
    )(diffs.reshape(128, 128))
    return loss[0, 0]
